# baseline (speedup 1.0000x reference)
import jax
import jax.numpy as jnp
from jax import lax
from jax.experimental import pallas as pl
from jax.experimental.pallas import tpu as pltpu
from jax.experimental.pallas import tpu_sc as plsc

NREL = 3
D_IN = 128
D_OUT = 128
DH = 64
DP = 80

NC = 2
NS = 16
CHUNK = 256
CHUNKS_PER_TILE = 50
EDGES_PER_TILE = CHUNK * CHUNKS_PER_TILE
E_PAD = EDGES_PER_TILE * NS
ACC_ROWS = 10240
STRIPE = ACC_ROWS // NS
NBUF = 2
LOOKAHEAD = 1


def _project_kernel(x_ref, w_ref, b_ref, out_ref):
    acc = jnp.dot(x_ref[...], w_ref[0], preferred_element_type=jnp.float32)
    acc = acc + b_ref[0]
    nb = acc.shape[0]
    tail = (lax.broadcasted_iota(jnp.int32, (nb, DP - DH), 1) == 0)
    tail = tail.astype(jnp.float32)
    out_ref[0, 0] = jnp.concatenate([acc[:, :DH], tail], axis=1)
    out_ref[0, 1] = jnp.concatenate([acc[:, DH:], tail], axis=1)


def _project(x, Wstk, bstk):
    n = x.shape[0]
    blk = 1000
    grid = (NREL, n // blk)
    return pl.pallas_call(
        _project_kernel,
        grid=grid,
        in_specs=[
            pl.BlockSpec((blk, D_IN), lambda r, i: (i, 0)),
            pl.BlockSpec((1, D_IN, D_OUT), lambda r, i: (r, 0, 0)),
            pl.BlockSpec((1, 1, D_OUT), lambda r, i: (r, 0, 0)),
        ],
        out_specs=pl.BlockSpec((1, NC, blk, DP), lambda r, i: (r, 0, i, 0)),
        out_shape=jax.ShapeDtypeStruct((NREL, NC, n, DP), jnp.float32),
    )(x, Wstk, bstk)


def _sc_body(wh0, wh1, wh2, s0, s1, s2, d0, d1, d2, zeros_ref, out,
             idx_src, idx_dst, rows, acc, sem_g, sem_s):
    c = lax.axis_index("c")
    s = lax.axis_index("s")
    wh_refs = (wh0, wh1, wh2)
    src_refs = (s0, s1, s2)
    dst_refs = (d0, d1, d2)
    tot = CHUNKS_PER_TILE
    for r in range(NREL):
        whr = wh_refs[r].at[c]
        pltpu.sync_copy(zeros_ref, acc.at[pl.ds(s * STRIPE, STRIPE)])
        pltpu.sync_copy(src_refs[r].at[s], idx_src)
        pltpu.sync_copy(dst_refs[r].at[s], idx_dst)
        plsc.subcore_barrier()

        def wait_g(j, b, whr=whr):
            pltpu.make_async_copy(whr.at[idx_src.at[j]], rows.at[b],
                                  sem_g).wait()

        def issue_g(j, b, whr=whr):
            pltpu.async_copy(whr.at[idx_src.at[j]], rows.at[b], sem_g)

        def issue_s(j, b):
            pltpu.async_copy(rows.at[b], acc.at[idx_dst.at[j]], sem_s,
                             add=True)

        def wait_s(j, b):
            pltpu.make_async_copy(rows.at[b], acc.at[idx_dst.at[j]],
                                  sem_s).wait()

        for j in range(LOOKAHEAD):
            issue_g(j, j % NBUF)
        for b in range(NBUF):
            wait_g(b, b)
            issue_s(b, b)
            if b >= NBUF - LOOKAHEAD:
                wait_s(b - (NBUF - LOOKAHEAD), (b + LOOKAHEAD) % NBUF)
            issue_g(b + LOOKAHEAD, (b + LOOKAHEAD) % NBUF)

        def steady(g, carry):
            j0 = g * NBUF
            for b in range(NBUF):
                wait_g(j0 + b, b)
                issue_s(j0 + b, b)
                b2 = (b + LOOKAHEAD) % NBUF
                wait_s(j0 + b - (NBUF - LOOKAHEAD), b2)
                issue_g(j0 + b + LOOKAHEAD, b2)
            return carry

        n_groups = tot // NBUF
        lax.fori_loop(1, n_groups - 1, steady, 0)
        j0 = (n_groups - 1) * NBUF
        for b in range(NBUF):
            wait_g(j0 + b, b)
            issue_s(j0 + b, b)
            b2 = (b + LOOKAHEAD) % NBUF
            if b < NBUF - LOOKAHEAD:
                wait_s(j0 + b - (NBUF - LOOKAHEAD), b2)
                issue_g(j0 + b + LOOKAHEAD, b2)
        for j in range(tot - NBUF, tot):
            wait_s(j, j % NBUF)

        plsc.subcore_barrier()
        pltpu.sync_copy(acc.at[pl.ds(s * STRIPE, STRIPE)],
                        out.at[r, c, pl.ds(s * STRIPE, STRIPE)])
        plsc.subcore_barrier()


def _sc_aggregate(wh, srcs, dsts, zeros):
    mesh = plsc.VectorSubcoreMesh(core_axis_name="c", subcore_axis_name="s")
    fn = pl.kernel(
        _sc_body,
        out_type=jax.ShapeDtypeStruct((NREL, NC, ACC_ROWS, DP), jnp.float32),
        mesh=mesh,
        scratch_types=[
            pltpu.VMEM((CHUNKS_PER_TILE, CHUNK), jnp.int32),
            pltpu.VMEM((CHUNKS_PER_TILE, CHUNK), jnp.int32),
            pltpu.VMEM((NBUF, CHUNK, DP), jnp.float32),
            pltpu.VMEM_SHARED((ACC_ROWS, DP), jnp.float32),
            pltpu.SemaphoreType.DMA,
            pltpu.SemaphoreType.DMA,
        ],
        compiler_params=pltpu.CompilerParams(use_tc_tiling_on_sc=False),
    )
    return fn(wh[0], wh[1], wh[2], srcs[0], srcs[1], srcs[2],
              dsts[0], dsts[1], dsts[2], zeros)


def _reduce_kernel(p_ref, out_ref):
    p = p_ref[...]
    nb = p.shape[2]
    halves = []
    for c in range(NC):
        h = jnp.zeros((nb, DH), jnp.float32)
        for r in range(NREL):
            cnt = p[r, c, :, DH:DH + 1]
            feats = p[r, c, :, :DH]
            inv = jnp.where(cnt > 0.0, 1.0 / jnp.maximum(cnt, 1.0), 0.0)
            h = h + feats * inv
        halves.append(h)
    out_ref[...] = jnp.concatenate(halves, axis=1)


def _reduce(partials, n):
    blk = 1000
    return pl.pallas_call(
        _reduce_kernel,
        grid=(n // blk,),
        in_specs=[pl.BlockSpec((NREL, NC, blk, DP), lambda i: (0, 0, i, 0))],
        out_specs=pl.BlockSpec((blk, D_OUT), lambda i: (i, 0)),
        out_shape=jax.ShapeDtypeStruct((n, D_OUT), jnp.float32),
    )(partials)


def kernel(x, edge_index_r0, edge_index_r1, edge_index_r2,
           W_r0, b_r0, W_r1, b_r1, W_r2, b_r2):
    n = x.shape[0]
    e = edge_index_r0.shape[1]
    Wstk = jnp.stack([W_r0, W_r1, W_r2])
    bstk = jnp.stack([b_r0, b_r1, b_r2]).reshape(NREL, 1, D_OUT)

    srcs, dsts = [], []
    pad = E_PAD - e
    for ei in (edge_index_r0, edge_index_r1, edge_index_r2):
        src = jnp.concatenate([ei[0], jnp.zeros((pad,), jnp.int32)])
        dst = jnp.concatenate([ei[1], jnp.full((pad,), n, jnp.int32)])
        srcs.append(src.reshape(NS, CHUNKS_PER_TILE, CHUNK))
        dsts.append(dst.reshape(NS, CHUNKS_PER_TILE, CHUNK))

    wh = _project(x, Wstk, bstk)
    zeros = jnp.zeros((STRIPE, DP), jnp.float32)
    partials = _sc_aggregate([wh[0], wh[1], wh[2]], srcs, dsts, zeros)
    return _reduce(partials, n)


# speedup vs baseline: 1.0353x; 1.0353x over previous
import jax
import jax.numpy as jnp
from jax import lax
from jax.experimental import pallas as pl
from jax.experimental.pallas import tpu as pltpu
from jax.experimental.pallas import tpu_sc as plsc

NREL = 3
D_IN = 128
D_OUT = 128
DH = 64
DP = 80

NC = 2
NS = 16
CHUNK = 128
CHUNKS_PER_TILE = 100
EDGES_PER_TILE = CHUNK * CHUNKS_PER_TILE
E_PAD = EDGES_PER_TILE * NS
ACC_ROWS = 10240
STRIPE = ACC_ROWS // NS
NBUF = 4
LOOKAHEAD = 2


def _project_kernel(x_ref, w_ref, b_ref, out_ref):
    acc = jnp.dot(x_ref[...], w_ref[0], preferred_element_type=jnp.float32)
    acc = acc + b_ref[0]
    nb = acc.shape[0]
    tail = (lax.broadcasted_iota(jnp.int32, (nb, DP - DH), 1) == 0)
    tail = tail.astype(jnp.float32)
    out_ref[0, 0] = jnp.concatenate([acc[:, :DH], tail], axis=1)
    out_ref[0, 1] = jnp.concatenate([acc[:, DH:], tail], axis=1)


def _project(x, Wstk, bstk):
    n = x.shape[0]
    blk = 1000
    grid = (NREL, n // blk)
    return pl.pallas_call(
        _project_kernel,
        grid=grid,
        in_specs=[
            pl.BlockSpec((blk, D_IN), lambda r, i: (i, 0)),
            pl.BlockSpec((1, D_IN, D_OUT), lambda r, i: (r, 0, 0)),
            pl.BlockSpec((1, 1, D_OUT), lambda r, i: (r, 0, 0)),
        ],
        out_specs=pl.BlockSpec((1, NC, blk, DP), lambda r, i: (r, 0, i, 0)),
        out_shape=jax.ShapeDtypeStruct((NREL, NC, n, DP), jnp.float32),
    )(x, Wstk, bstk)


def _sc_body(wh0, wh1, wh2, s0, s1, s2, d0, d1, d2, zeros_ref, out,
             idx_src, idx_dst, rows, acc, sem_g, sem_s):
    c = lax.axis_index("c")
    s = lax.axis_index("s")
    wh_refs = (wh0, wh1, wh2)
    src_refs = (s0, s1, s2)
    dst_refs = (d0, d1, d2)
    tot = CHUNKS_PER_TILE
    for r in range(NREL):
        whr = wh_refs[r].at[c]
        pltpu.sync_copy(zeros_ref, acc.at[pl.ds(s * STRIPE, STRIPE)])
        pltpu.sync_copy(src_refs[r].at[s], idx_src)
        pltpu.sync_copy(dst_refs[r].at[s], idx_dst)
        plsc.subcore_barrier()

        def wait_g(j, b, whr=whr):
            pltpu.make_async_copy(whr.at[idx_src.at[j]], rows.at[b],
                                  sem_g).wait()

        def issue_g(j, b, whr=whr):
            pltpu.async_copy(whr.at[idx_src.at[j]], rows.at[b], sem_g)

        def issue_s(j, b):
            pltpu.async_copy(rows.at[b], acc.at[idx_dst.at[j]], sem_s,
                             add=True)

        def wait_s(j, b):
            pltpu.make_async_copy(rows.at[b], acc.at[idx_dst.at[j]],
                                  sem_s).wait()

        for j in range(LOOKAHEAD):
            issue_g(j, j % NBUF)
        for b in range(NBUF):
            wait_g(b, b)
            issue_s(b, b)
            if b >= NBUF - LOOKAHEAD:
                wait_s(b - (NBUF - LOOKAHEAD), (b + LOOKAHEAD) % NBUF)
            issue_g(b + LOOKAHEAD, (b + LOOKAHEAD) % NBUF)

        def steady(g, carry):
            j0 = g * NBUF
            for b in range(NBUF):
                wait_g(j0 + b, b)
                issue_s(j0 + b, b)
                b2 = (b + LOOKAHEAD) % NBUF
                wait_s(j0 + b - (NBUF - LOOKAHEAD), b2)
                issue_g(j0 + b + LOOKAHEAD, b2)
            return carry

        n_groups = tot // NBUF
        lax.fori_loop(1, n_groups - 1, steady, 0)
        j0 = (n_groups - 1) * NBUF
        for b in range(NBUF):
            wait_g(j0 + b, b)
            issue_s(j0 + b, b)
            b2 = (b + LOOKAHEAD) % NBUF
            if b < NBUF - LOOKAHEAD:
                wait_s(j0 + b - (NBUF - LOOKAHEAD), b2)
                issue_g(j0 + b + LOOKAHEAD, b2)
        for j in range(tot - NBUF, tot):
            wait_s(j, j % NBUF)

        plsc.subcore_barrier()
        pltpu.sync_copy(acc.at[pl.ds(s * STRIPE, STRIPE)],
                        out.at[r, c, pl.ds(s * STRIPE, STRIPE)])
        plsc.subcore_barrier()


def _sc_aggregate(wh, srcs, dsts, zeros):
    mesh = plsc.VectorSubcoreMesh(core_axis_name="c", subcore_axis_name="s")
    fn = pl.kernel(
        _sc_body,
        out_type=jax.ShapeDtypeStruct((NREL, NC, ACC_ROWS, DP), jnp.float32),
        mesh=mesh,
        scratch_types=[
            pltpu.VMEM((CHUNKS_PER_TILE, CHUNK), jnp.int32),
            pltpu.VMEM((CHUNKS_PER_TILE, CHUNK), jnp.int32),
            pltpu.VMEM((NBUF, CHUNK, DP), jnp.float32),
            pltpu.VMEM_SHARED((ACC_ROWS, DP), jnp.float32),
            pltpu.SemaphoreType.DMA,
            pltpu.SemaphoreType.DMA,
        ],
        compiler_params=pltpu.CompilerParams(use_tc_tiling_on_sc=False),
    )
    return fn(wh[0], wh[1], wh[2], srcs[0], srcs[1], srcs[2],
              dsts[0], dsts[1], dsts[2], zeros)


def _reduce_kernel(p_ref, out_ref):
    p = p_ref[...]
    nb = p.shape[2]
    halves = []
    for c in range(NC):
        h = jnp.zeros((nb, DH), jnp.float32)
        for r in range(NREL):
            cnt = p[r, c, :, DH:DH + 1]
            feats = p[r, c, :, :DH]
            inv = jnp.where(cnt > 0.0, 1.0 / jnp.maximum(cnt, 1.0), 0.0)
            h = h + feats * inv
        halves.append(h)
    out_ref[...] = jnp.concatenate(halves, axis=1)


def _reduce(partials, n):
    blk = 1000
    return pl.pallas_call(
        _reduce_kernel,
        grid=(n // blk,),
        in_specs=[pl.BlockSpec((NREL, NC, blk, DP), lambda i: (0, 0, i, 0))],
        out_specs=pl.BlockSpec((blk, D_OUT), lambda i: (i, 0)),
        out_shape=jax.ShapeDtypeStruct((n, D_OUT), jnp.float32),
    )(partials)


def kernel(x, edge_index_r0, edge_index_r1, edge_index_r2,
           W_r0, b_r0, W_r1, b_r1, W_r2, b_r2):
    n = x.shape[0]
    e = edge_index_r0.shape[1]
    Wstk = jnp.stack([W_r0, W_r1, W_r2])
    bstk = jnp.stack([b_r0, b_r1, b_r2]).reshape(NREL, 1, D_OUT)

    srcs, dsts = [], []
    pad = E_PAD - e
    for ei in (edge_index_r0, edge_index_r1, edge_index_r2):
        src = jnp.concatenate([ei[0], jnp.zeros((pad,), jnp.int32)])
        dst = jnp.concatenate([ei[1], jnp.full((pad,), n, jnp.int32)])
        srcs.append(src.reshape(NS, CHUNKS_PER_TILE, CHUNK))
        dsts.append(dst.reshape(NS, CHUNKS_PER_TILE, CHUNK))

    wh = _project(x, Wstk, bstk)
    zeros = jnp.zeros((STRIPE, DP), jnp.float32)
    partials = _sc_aggregate([wh[0], wh[1], wh[2]], srcs, dsts, zeros)
    return _reduce(partials, n)


# speedup vs baseline: 1.2234x; 1.1817x over previous
import jax
import jax.numpy as jnp
from jax import lax
from jax.experimental import pallas as pl
from jax.experimental.pallas import tpu as pltpu
from jax.experimental.pallas import tpu_sc as plsc

NREL = 3
D_IN = 128
D_OUT = 128
DH = 64
DP = 64
DC = 16

NC = 2
NS = 16
CHUNK = 128
CHUNKS_PER_TILE = 100
EDGES_PER_TILE = CHUNK * CHUNKS_PER_TILE
E_PAD = EDGES_PER_TILE * NS
ACC_ROWS = 10240
STRIPE = ACC_ROWS // NS
NBUF = 4
LOOKAHEAD = 2


def _project_kernel(x_ref, w_ref, b_ref, out_ref):
    acc = jnp.dot(x_ref[...], w_ref[0], preferred_element_type=jnp.float32)
    acc = acc + b_ref[0]
    out_ref[0, 0] = acc[:, :DH]
    out_ref[0, 1] = acc[:, DH:]


def _project(x, Wstk, bstk):
    n = x.shape[0]
    blk = 1000
    grid = (NREL, n // blk)
    return pl.pallas_call(
        _project_kernel,
        grid=grid,
        in_specs=[
            pl.BlockSpec((blk, D_IN), lambda r, i: (i, 0)),
            pl.BlockSpec((1, D_IN, D_OUT), lambda r, i: (r, 0, 0)),
            pl.BlockSpec((1, 1, D_OUT), lambda r, i: (r, 0, 0)),
        ],
        out_specs=pl.BlockSpec((1, NC, blk, DP), lambda r, i: (r, 0, i, 0)),
        out_shape=jax.ShapeDtypeStruct((NREL, NC, n, DP), jnp.float32),
    )(x, Wstk, bstk)


def _sc_body(wh0, wh1, wh2, s0, s1, s2, d0, d1, d2, zeros_ref, zc_ref,
             ones_ref, out, out_cnt,
             idx_src, idx_dst, rows, ones, acc, cacc, sem_g, sem_s, sem_c):
    c = lax.axis_index("c")
    s = lax.axis_index("s")
    wh_refs = (wh0, wh1, wh2)
    src_refs = (s0, s1, s2)
    dst_refs = (d0, d1, d2)
    tot = CHUNKS_PER_TILE
    pltpu.sync_copy(ones_ref, ones.at[0])
    for r in range(NREL):
        whr = wh_refs[r].at[c]
        pltpu.sync_copy(zeros_ref, acc.at[pl.ds(s * STRIPE, STRIPE)])
        pltpu.sync_copy(zc_ref, cacc.at[pl.ds(s * STRIPE, STRIPE)])
        pltpu.sync_copy(src_refs[r].at[s], idx_src)
        pltpu.sync_copy(dst_refs[r].at[s], idx_dst)
        plsc.subcore_barrier()

        def wait_g(j, b, whr=whr):
            pltpu.make_async_copy(whr.at[idx_src.at[j]], rows.at[b],
                                  sem_g).wait()

        def issue_g(j, b, whr=whr):
            pltpu.async_copy(whr.at[idx_src.at[j]], rows.at[b], sem_g)

        def issue_s(j, b):
            pltpu.async_copy(rows.at[b], acc.at[idx_dst.at[j]], sem_s,
                             add=True)
            pltpu.async_copy(ones.at[0], cacc.at[idx_dst.at[j]], sem_c,
                             add=True)

        def wait_s(j, b):
            pltpu.make_async_copy(rows.at[b], acc.at[idx_dst.at[j]],
                                  sem_s).wait()
            pltpu.make_async_copy(ones.at[0], cacc.at[idx_dst.at[j]],
                                  sem_c).wait()

        for j in range(LOOKAHEAD):
            issue_g(j, j % NBUF)
        for b in range(NBUF):
            wait_g(b, b)
            issue_s(b, b)
            if b >= NBUF - LOOKAHEAD:
                wait_s(b - (NBUF - LOOKAHEAD), (b + LOOKAHEAD) % NBUF)
            issue_g(b + LOOKAHEAD, (b + LOOKAHEAD) % NBUF)

        def steady(g, carry):
            j0 = g * NBUF
            for b in range(NBUF):
                wait_g(j0 + b, b)
                issue_s(j0 + b, b)
                b2 = (b + LOOKAHEAD) % NBUF
                wait_s(j0 + b - (NBUF - LOOKAHEAD), b2)
                issue_g(j0 + b + LOOKAHEAD, b2)
            return carry

        n_groups = tot // NBUF
        lax.fori_loop(1, n_groups - 1, steady, 0)
        j0 = (n_groups - 1) * NBUF
        for b in range(NBUF):
            wait_g(j0 + b, b)
            issue_s(j0 + b, b)
            b2 = (b + LOOKAHEAD) % NBUF
            if b < NBUF - LOOKAHEAD:
                wait_s(j0 + b - (NBUF - LOOKAHEAD), b2)
                issue_g(j0 + b + LOOKAHEAD, b2)
        for j in range(tot - NBUF, tot):
            wait_s(j, j % NBUF)

        plsc.subcore_barrier()
        pltpu.sync_copy(acc.at[pl.ds(s * STRIPE, STRIPE)],
                        out.at[r, c, pl.ds(s * STRIPE, STRIPE)])
        pltpu.sync_copy(cacc.at[pl.ds(s * STRIPE, STRIPE)],
                        out_cnt.at[r, c, pl.ds(s * STRIPE, STRIPE)])
        plsc.subcore_barrier()


def _sc_aggregate(wh, srcs, dsts, zeros, zeros_c, ones):
    mesh = plsc.VectorSubcoreMesh(core_axis_name="c", subcore_axis_name="s")
    fn = pl.kernel(
        _sc_body,
        out_type=(
            jax.ShapeDtypeStruct((NREL, NC, ACC_ROWS, DP), jnp.float32),
            jax.ShapeDtypeStruct((NREL, NC, ACC_ROWS, DC), jnp.float32),
        ),
        mesh=mesh,
        scratch_types=[
            pltpu.VMEM((CHUNKS_PER_TILE, CHUNK), jnp.int32),
            pltpu.VMEM((CHUNKS_PER_TILE, CHUNK), jnp.int32),
            pltpu.VMEM((NBUF, CHUNK, DP), jnp.float32),
            pltpu.VMEM((1, CHUNK, DC), jnp.float32),
            pltpu.VMEM_SHARED((ACC_ROWS, DP), jnp.float32),
            pltpu.VMEM_SHARED((ACC_ROWS, DC), jnp.float32),
            pltpu.SemaphoreType.DMA,
            pltpu.SemaphoreType.DMA,
            pltpu.SemaphoreType.DMA,
        ],
        compiler_params=pltpu.CompilerParams(use_tc_tiling_on_sc=False),
    )
    return fn(wh[0], wh[1], wh[2], srcs[0], srcs[1], srcs[2],
              dsts[0], dsts[1], dsts[2], zeros, zeros_c, ones)


def _reduce_kernel(p_ref, c_ref, out_ref):
    p = p_ref[...]
    cc = c_ref[...]
    nb = p.shape[2]
    halves = []
    for c in range(NC):
        h = jnp.zeros((nb, DH), jnp.float32)
        for r in range(NREL):
            cnt = cc[r, c, :, 0:1]
            feats = p[r, c, :, :DH]
            inv = jnp.where(cnt > 0.0, 1.0 / jnp.maximum(cnt, 1.0), 0.0)
            h = h + feats * inv
        halves.append(h)
    out_ref[...] = jnp.concatenate(halves, axis=1)


def _reduce(partials, counts, n):
    blk = 1000
    return pl.pallas_call(
        _reduce_kernel,
        grid=(n // blk,),
        in_specs=[
            pl.BlockSpec((NREL, NC, blk, DP), lambda i: (0, 0, i, 0)),
            pl.BlockSpec((NREL, NC, blk, DC), lambda i: (0, 0, i, 0)),
        ],
        out_specs=pl.BlockSpec((blk, D_OUT), lambda i: (i, 0)),
        out_shape=jax.ShapeDtypeStruct((n, D_OUT), jnp.float32),
    )(partials, counts)


def kernel(x, edge_index_r0, edge_index_r1, edge_index_r2,
           W_r0, b_r0, W_r1, b_r1, W_r2, b_r2):
    n = x.shape[0]
    e = edge_index_r0.shape[1]
    Wstk = jnp.stack([W_r0, W_r1, W_r2])
    bstk = jnp.stack([b_r0, b_r1, b_r2]).reshape(NREL, 1, D_OUT)

    srcs, dsts = [], []
    pad = E_PAD - e
    for ei in (edge_index_r0, edge_index_r1, edge_index_r2):
        src = jnp.concatenate([ei[0], jnp.zeros((pad,), jnp.int32)])
        dst = jnp.concatenate([ei[1], jnp.full((pad,), n, jnp.int32)])
        srcs.append(src.reshape(NS, CHUNKS_PER_TILE, CHUNK))
        dsts.append(dst.reshape(NS, CHUNKS_PER_TILE, CHUNK))

    wh = _project(x, Wstk, bstk)
    zeros = jnp.zeros((STRIPE, DP), jnp.float32)
    zeros_c = jnp.zeros((STRIPE, DC), jnp.float32)
    ones = jnp.ones((CHUNK, DC), jnp.float32)
    partials, counts = _sc_aggregate([wh[0], wh[1], wh[2]], srcs, dsts,
                                     zeros, zeros_c, ones)
    return _reduce(partials, counts, n)


# speedup vs baseline: 1.2424x; 1.0155x over previous
import jax
import jax.numpy as jnp
from jax import lax
from jax.experimental import pallas as pl
from jax.experimental.pallas import tpu as pltpu
from jax.experimental.pallas import tpu_sc as plsc

NREL = 3
D_IN = 128
D_OUT = 128
DH = 64
DP = 64
DC = 16

NC = 2
NS = 16
CHUNK = 128
CHUNKS_PER_TILE = 100
EDGES_PER_TILE = CHUNK * CHUNKS_PER_TILE
E_PAD = EDGES_PER_TILE * NS
ACC_ROWS = 10240
STRIPE = ACC_ROWS // NS
NBUF = 5
LOOKAHEAD = 3


def _project_kernel(x_ref, w_ref, b_ref, out_ref):
    acc = jnp.dot(x_ref[...], w_ref[0], preferred_element_type=jnp.float32)
    acc = acc + b_ref[0]
    out_ref[0, 0] = acc[:, :DH]
    out_ref[0, 1] = acc[:, DH:]


def _project(x, Wstk, bstk):
    n = x.shape[0]
    blk = 1000
    grid = (NREL, n // blk)
    return pl.pallas_call(
        _project_kernel,
        grid=grid,
        in_specs=[
            pl.BlockSpec((blk, D_IN), lambda r, i: (i, 0)),
            pl.BlockSpec((1, D_IN, D_OUT), lambda r, i: (r, 0, 0)),
            pl.BlockSpec((1, 1, D_OUT), lambda r, i: (r, 0, 0)),
        ],
        out_specs=pl.BlockSpec((1, NC, blk, DP), lambda r, i: (r, 0, i, 0)),
        out_shape=jax.ShapeDtypeStruct((NREL, NC, n, DP), jnp.float32),
    )(x, Wstk, bstk)


def _sc_body(wh0, wh1, wh2, s0, s1, s2, d0, d1, d2, zeros_ref, zc_ref,
             ones_ref, out, out_cnt,
             idx_src, idx_dst, rows, ones, acc, cacc, sem_g, sem_s, sem_c):
    c = lax.axis_index("c")
    s = lax.axis_index("s")
    wh_refs = (wh0, wh1, wh2)
    src_refs = (s0, s1, s2)
    dst_refs = (d0, d1, d2)
    tot = CHUNKS_PER_TILE
    pltpu.sync_copy(ones_ref, ones.at[0])
    for r in range(NREL):
        whr = wh_refs[r].at[c]
        pltpu.sync_copy(zeros_ref, acc.at[pl.ds(s * STRIPE, STRIPE)])
        pltpu.sync_copy(zc_ref, cacc.at[pl.ds(s * STRIPE, STRIPE)])
        pltpu.sync_copy(src_refs[r].at[s], idx_src)
        pltpu.sync_copy(dst_refs[r].at[s], idx_dst)
        plsc.subcore_barrier()

        def wait_g(j, b, whr=whr):
            pltpu.make_async_copy(whr.at[idx_src.at[j]], rows.at[b],
                                  sem_g).wait()

        def issue_g(j, b, whr=whr):
            pltpu.async_copy(whr.at[idx_src.at[j]], rows.at[b], sem_g)

        def issue_s(j, b):
            pltpu.async_copy(rows.at[b], acc.at[idx_dst.at[j]], sem_s,
                             add=True)
            pltpu.async_copy(ones.at[0], cacc.at[idx_dst.at[j]], sem_c,
                             add=True)

        def wait_s(j, b):
            pltpu.make_async_copy(rows.at[b], acc.at[idx_dst.at[j]],
                                  sem_s).wait()
            pltpu.make_async_copy(ones.at[0], cacc.at[idx_dst.at[j]],
                                  sem_c).wait()

        for j in range(LOOKAHEAD):
            issue_g(j, j % NBUF)
        for b in range(NBUF):
            wait_g(b, b)
            issue_s(b, b)
            if b >= NBUF - LOOKAHEAD:
                wait_s(b - (NBUF - LOOKAHEAD), (b + LOOKAHEAD) % NBUF)
            issue_g(b + LOOKAHEAD, (b + LOOKAHEAD) % NBUF)

        def steady(g, carry):
            j0 = g * NBUF
            for b in range(NBUF):
                wait_g(j0 + b, b)
                issue_s(j0 + b, b)
                b2 = (b + LOOKAHEAD) % NBUF
                wait_s(j0 + b - (NBUF - LOOKAHEAD), b2)
                issue_g(j0 + b + LOOKAHEAD, b2)
            return carry

        n_groups = tot // NBUF
        lax.fori_loop(1, n_groups - 1, steady, 0)
        j0 = (n_groups - 1) * NBUF
        for b in range(NBUF):
            wait_g(j0 + b, b)
            issue_s(j0 + b, b)
            b2 = (b + LOOKAHEAD) % NBUF
            if b < NBUF - LOOKAHEAD:
                wait_s(j0 + b - (NBUF - LOOKAHEAD), b2)
                issue_g(j0 + b + LOOKAHEAD, b2)
        for j in range(tot - NBUF, tot):
            wait_s(j, j % NBUF)

        plsc.subcore_barrier()
        pltpu.sync_copy(acc.at[pl.ds(s * STRIPE, STRIPE)],
                        out.at[r, c, pl.ds(s * STRIPE, STRIPE)])
        pltpu.sync_copy(cacc.at[pl.ds(s * STRIPE, STRIPE)],
                        out_cnt.at[r, c, pl.ds(s * STRIPE, STRIPE)])
        plsc.subcore_barrier()


def _sc_aggregate(wh, srcs, dsts, zeros, zeros_c, ones):
    mesh = plsc.VectorSubcoreMesh(core_axis_name="c", subcore_axis_name="s")
    fn = pl.kernel(
        _sc_body,
        out_type=(
            jax.ShapeDtypeStruct((NREL, NC, ACC_ROWS, DP), jnp.float32),
            jax.ShapeDtypeStruct((NREL, NC, ACC_ROWS, DC), jnp.float32),
        ),
        mesh=mesh,
        scratch_types=[
            pltpu.VMEM((CHUNKS_PER_TILE, CHUNK), jnp.int32),
            pltpu.VMEM((CHUNKS_PER_TILE, CHUNK), jnp.int32),
            pltpu.VMEM((NBUF, CHUNK, DP), jnp.float32),
            pltpu.VMEM((1, CHUNK, DC), jnp.float32),
            pltpu.VMEM_SHARED((ACC_ROWS, DP), jnp.float32),
            pltpu.VMEM_SHARED((ACC_ROWS, DC), jnp.float32),
            pltpu.SemaphoreType.DMA,
            pltpu.SemaphoreType.DMA,
            pltpu.SemaphoreType.DMA,
        ],
        compiler_params=pltpu.CompilerParams(use_tc_tiling_on_sc=False),
    )
    return fn(wh[0], wh[1], wh[2], srcs[0], srcs[1], srcs[2],
              dsts[0], dsts[1], dsts[2], zeros, zeros_c, ones)


def _reduce_kernel(p_ref, c_ref, out_ref):
    p = p_ref[...]
    cc = c_ref[...]
    nb = p.shape[2]
    halves = []
    for c in range(NC):
        h = jnp.zeros((nb, DH), jnp.float32)
        for r in range(NREL):
            cnt = cc[r, c, :, 0:1]
            feats = p[r, c, :, :DH]
            inv = jnp.where(cnt > 0.0, 1.0 / jnp.maximum(cnt, 1.0), 0.0)
            h = h + feats * inv
        halves.append(h)
    out_ref[...] = jnp.concatenate(halves, axis=1)


def _reduce(partials, counts, n):
    blk = 1000
    return pl.pallas_call(
        _reduce_kernel,
        grid=(n // blk,),
        in_specs=[
            pl.BlockSpec((NREL, NC, blk, DP), lambda i: (0, 0, i, 0)),
            pl.BlockSpec((NREL, NC, blk, DC), lambda i: (0, 0, i, 0)),
        ],
        out_specs=pl.BlockSpec((blk, D_OUT), lambda i: (i, 0)),
        out_shape=jax.ShapeDtypeStruct((n, D_OUT), jnp.float32),
    )(partials, counts)


def kernel(x, edge_index_r0, edge_index_r1, edge_index_r2,
           W_r0, b_r0, W_r1, b_r1, W_r2, b_r2):
    n = x.shape[0]
    e = edge_index_r0.shape[1]
    Wstk = jnp.stack([W_r0, W_r1, W_r2])
    bstk = jnp.stack([b_r0, b_r1, b_r2]).reshape(NREL, 1, D_OUT)

    srcs, dsts = [], []
    pad = E_PAD - e
    for ei in (edge_index_r0, edge_index_r1, edge_index_r2):
        src = jnp.concatenate([ei[0], jnp.zeros((pad,), jnp.int32)])
        dst = jnp.concatenate([ei[1], jnp.full((pad,), n, jnp.int32)])
        srcs.append(src.reshape(NS, CHUNKS_PER_TILE, CHUNK))
        dsts.append(dst.reshape(NS, CHUNKS_PER_TILE, CHUNK))

    wh = _project(x, Wstk, bstk)
    zeros = jnp.zeros((STRIPE, DP), jnp.float32)
    zeros_c = jnp.zeros((STRIPE, DC), jnp.float32)
    ones = jnp.ones((CHUNK, DC), jnp.float32)
    partials, counts = _sc_aggregate([wh[0], wh[1], wh[2]], srcs, dsts,
                                     zeros, zeros_c, ones)
    return _reduce(partials, counts, n)
